# baseline (speedup 1.0000x reference)
import functools

import jax
import jax.numpy as jnp
from jax import lax
from jax.experimental import pallas as pl
from jax.experimental.pallas import tpu as pltpu
from jax.experimental.pallas import tpu_sc as plsc

_NSC = 2
_NSUB = 16
_NW = _NSC * _NSUB
_CW = 125
_ROW_BLK = 2000

_mesh = plsc.VectorSubcoreMesh(core_axis_name="c", subcore_axis_name="s")
_SC_PARAMS = pltpu.CompilerParams(use_tc_tiling_on_sc=False)


_HW = 16


def _sc_degrees(nidx, hidx, ones_col, z1):
    nw, nch, cw = nidx.shape
    n = z1.shape[0]
    rps = (n // _NSUB) & ~7
    tail = n - rps * _NSUB

    @functools.partial(
        pl.kernel,
        out_type=(jax.ShapeDtypeStruct((_NSC, n, _HW), jnp.float32),
                  jax.ShapeDtypeStruct((_NSC, n, _HW), jnp.float32)),
        mesh=_mesh,
        scratch_types=[
            pltpu.VMEM((nch, cw), jnp.int32),
            pltpu.VMEM((nch, cw), jnp.int32),
            pltpu.VMEM((cw, _HW), jnp.float32),
            pltpu.VMEM_SHARED((n, _HW), jnp.float32),
            pltpu.VMEM_SHARED((n, _HW), jnp.float32),
        ],
        compiler_params=_SC_PARAMS,
    )
    def run(n_hbm, h_hbm, ones_hbm, z_hbm, dv_hbm, de_hbm,
            n_v, h_v, ones_v, dv_acc, de_acc):
        c = lax.axis_index("c")
        s = lax.axis_index("s")
        w = c * _NSUB + s
        pltpu.sync_copy(n_hbm.at[w], n_v)
        pltpu.sync_copy(h_hbm.at[w], h_v)
        pltpu.sync_copy(ones_hbm, ones_v)
        sl = pl.ds(s * rps, rps)
        pltpu.sync_copy(z_hbm.at[sl], dv_acc.at[sl])
        pltpu.sync_copy(z_hbm.at[sl], de_acc.at[sl])
        if tail:
            tl = pl.ds(rps * _NSUB, tail)

            @pl.when(s == 0)
            def _():
                pltpu.sync_copy(z_hbm.at[tl], dv_acc.at[tl])
                pltpu.sync_copy(z_hbm.at[tl], de_acc.at[tl])
        plsc.subcore_barrier()

        @pl.loop(0, nch)
        def _(j):
            pltpu.sync_copy(ones_v, dv_acc.at[n_v.at[j]], add=True)
            pltpu.sync_copy(ones_v, de_acc.at[h_v.at[j]], add=True)

        plsc.subcore_barrier()
        pltpu.sync_copy(dv_acc.at[sl], dv_hbm.at[c, sl])
        pltpu.sync_copy(de_acc.at[sl], de_hbm.at[c, sl])
        if tail:
            tl2 = pl.ds(rps * _NSUB, tail)

            @pl.when(s == 0)
            def _():
                pltpu.sync_copy(dv_acc.at[tl2], dv_hbm.at[c, tl2])
                pltpu.sync_copy(de_acc.at[tl2], de_hbm.at[c, tl2])

    return run(nidx, hidx, ones_col, z1)


def _sc_scatter_pass(tab, gidx, sidx, zeros):
    n, d = tab.shape
    nw, nch, cw = gidx.shape
    rps = (n // _NSUB) & ~7
    tail = n - rps * _NSUB
    nh = nch // 2
    assert nh * 2 == nch and nh % 2 == 0

    @functools.partial(
        pl.kernel,
        out_type=jax.ShapeDtypeStruct((_NSC, n, d), jnp.float32),
        mesh=_mesh,
        scratch_types=[
            pltpu.VMEM((nh, cw), jnp.int32),
            pltpu.VMEM((nh, cw), jnp.int32),
            pltpu.VMEM((cw, d), jnp.float32),
            pltpu.VMEM((cw, d), jnp.float32),
            pltpu.VMEM_SHARED((n, d), jnp.float32),
            pltpu.SemaphoreType.DMA,
            pltpu.SemaphoreType.DMA,
            pltpu.SemaphoreType.DMA,
            pltpu.SemaphoreType.DMA,
        ],
        compiler_params=_SC_PARAMS,
    )
    def run(tab_hbm, g_hbm, s_hbm, z_hbm, out_hbm,
            g_v, s_v, buf0, buf1, acc, gsem0, gsem1, ssem0, ssem1):
        c = lax.axis_index("c")
        s = lax.axis_index("s")
        w = c * _NSUB + s
        sl = pl.ds(s * rps, rps)
        pltpu.sync_copy(z_hbm.at[sl], acc.at[sl])
        if tail:
            tl = pl.ds(rps * _NSUB, tail)

            @pl.when(s == 0)
            def _():
                pltpu.sync_copy(z_hbm.at[tl], acc.at[tl])
        plsc.subcore_barrier()

        dummy = tab_hbm.at[pl.ds(0, cw)]
        for h in range(2):
            pltpu.sync_copy(g_hbm.at[w, pl.ds(h * nh, nh)], g_v)
            pltpu.sync_copy(s_hbm.at[w, pl.ds(h * nh, nh)], s_v)
            pltpu.async_copy(tab_hbm.at[g_v.at[0]], buf0, gsem0)
            pltpu.async_copy(tab_hbm.at[g_v.at[1]], buf1, gsem1)

            @pl.loop(0, nh, step=2)
            def _(j):
                pltpu.make_async_copy(dummy, buf0, gsem0).wait()
                pltpu.async_copy(buf0, acc.at[s_v.at[j]], ssem0, add=True)
                pltpu.make_async_copy(dummy, buf1, gsem1).wait()
                pltpu.async_copy(buf1, acc.at[s_v.at[j + 1]], ssem1,
                                 add=True)

                @pl.when(j + 2 < nh)
                def _():
                    pltpu.make_async_copy(dummy, buf0, ssem0).wait()
                    pltpu.async_copy(tab_hbm.at[g_v.at[j + 2]], buf0, gsem0)

                @pl.when(j + 3 < nh)
                def _():
                    pltpu.make_async_copy(dummy, buf1, ssem1).wait()
                    pltpu.async_copy(tab_hbm.at[g_v.at[j + 3]], buf1, gsem1)

            pltpu.make_async_copy(dummy, buf0, ssem0).wait()
            pltpu.make_async_copy(dummy, buf1, ssem1).wait()

        plsc.subcore_barrier()
        pltpu.sync_copy(acc.at[sl], out_hbm.at[c, sl])
        if tail:
            tl2 = pl.ds(rps * _NSUB, tail)

            @pl.when(s == 0)
            def _():
                pltpu.sync_copy(acc.at[tl2], out_hbm.at[c, tl2])

    return run(tab, gidx, sidx, zeros)


def _lin1_body(x_ref, w_ref, b_ref, o_ref):
    o_ref[...] = (jnp.dot(x_ref[...], w_ref[...],
                          preferred_element_type=jnp.float32) + b_ref[...])


def _tc_linear1(x, w, b):
    n, din = x.shape
    dh = w.shape[1]
    return pl.pallas_call(
        _lin1_body,
        grid=(n // _ROW_BLK,),
        in_specs=[pl.BlockSpec((_ROW_BLK, din), lambda i: (i, 0)),
                  pl.BlockSpec((din, dh), lambda i: (0, 0)),
                  pl.BlockSpec((1, dh), lambda i: (0, 0))],
        out_specs=pl.BlockSpec((_ROW_BLK, dh), lambda i: (i, 0)),
        out_shape=jax.ShapeDtypeStruct((n, dh), jnp.float32),
    )(x, w, b)


def _deg_body(m_ref, dv_ref, de_ref, h_ref, dvis_ref, dei_ref):
    dv = dv_ref[0, :, :1] + dv_ref[1, :, :1]
    dvis = jnp.where(dv > 0, lax.rsqrt(jnp.maximum(dv, 1e-12)), 0.0)
    de = de_ref[0, :, :1] + de_ref[1, :, :1]
    dei = jnp.where(de > 0, 1.0 / jnp.maximum(de, 1e-12), 0.0)
    dvis_ref[...] = dvis
    dei_ref[...] = dei
    h_ref[...] = m_ref[...] * dvis


def _tc_degree_scale(m1, dv_p, de_p):
    n, dh = m1.shape
    col = lambda i: (i, 0)
    pspec = pl.BlockSpec((_NSC, _ROW_BLK, _HW), lambda i: (0, i, 0))
    cspec = pl.BlockSpec((_ROW_BLK, 1), col)
    return pl.pallas_call(
        _deg_body,
        grid=(n // _ROW_BLK,),
        in_specs=[pl.BlockSpec((_ROW_BLK, dh), col), pspec, pspec],
        out_specs=[pl.BlockSpec((_ROW_BLK, dh), col), cspec, cspec],
        out_shape=[jax.ShapeDtypeStruct((n, dh), jnp.float32),
                   jax.ShapeDtypeStruct((n, 1), jnp.float32),
                   jax.ShapeDtypeStruct((n, 1), jnp.float32)],
    )(m1, dv_p, de_p)


def _comb_body(p_ref, sc_ref, o_ref):
    o_ref[...] = (p_ref[0] + p_ref[1]) * sc_ref[...]


def _tc_combine(p, scale, dout=None):
    _, n, d = p.shape
    do = d if dout is None else dout
    col = lambda i: (i, 0)

    def body(p_ref, sc_ref, o_ref):
        o_ref[...] = ((p_ref[0] + p_ref[1]) * sc_ref[...])[:, :do]

    return pl.pallas_call(
        body,
        grid=(n // _ROW_BLK,),
        in_specs=[pl.BlockSpec((_NSC, _ROW_BLK, d), lambda i: (0, i, 0)),
                  pl.BlockSpec((_ROW_BLK, 1), col)],
        out_specs=pl.BlockSpec((_ROW_BLK, do), col),
        out_shape=jax.ShapeDtypeStruct((n, do), jnp.float32),
    )(p, scale)


def _layer2_body(p_ref, dvis_ref, w_ref, b_ref, o_ref):
    t = jax.nn.relu((p_ref[0] + p_ref[1]) * dvis_ref[...])
    o_ref[...] = (jnp.dot(t, w_ref[...], preferred_element_type=jnp.float32)
                  + b_ref[...]) * dvis_ref[...]


def _tc_layer2(p, dvis, w2p, b2p):
    _, n, dh = p.shape
    d2 = w2p.shape[1]
    col = lambda i: (i, 0)
    return pl.pallas_call(
        _layer2_body,
        grid=(n // _ROW_BLK,),
        in_specs=[pl.BlockSpec((_NSC, _ROW_BLK, dh), lambda i: (0, i, 0)),
                  pl.BlockSpec((_ROW_BLK, 1), col),
                  pl.BlockSpec((dh, d2), lambda i: (0, 0)),
                  pl.BlockSpec((1, d2), lambda i: (0, 0))],
        out_specs=pl.BlockSpec((_ROW_BLK, d2), col),
        out_shape=jax.ShapeDtypeStruct((n, d2), jnp.float32),
    )(p, dvis, w2p, b2p)


def kernel(x, hyperedge_index, W1, b1, W2, b2):
    n, din = x.shape
    dh = W1.shape[1]
    nc = W2.shape[1]
    e = hyperedge_index.shape[1]
    per = e // _NW
    nch = per // _CW
    assert per * _NW == e and nch * _CW == per and nch % 2 == 0

    node_rs = hyperedge_index[0].reshape(_NW, nch, _CW)
    he_rs = hyperedge_index[1].reshape(_NW, nch, _CW)

    d2 = 48
    z128 = jnp.zeros((n, dh), jnp.float32)
    z48 = jnp.zeros((n, d2), jnp.float32)
    z1 = jnp.zeros((n, _HW), jnp.float32)
    ones_col = jnp.ones((_CW, _HW), jnp.float32)
    w2p = jnp.pad(W2, ((0, 0), (0, d2 - nc)))
    b2p = jnp.pad(b2, (0, d2 - nc)).reshape(1, d2)

    dv_p, de_p = _sc_degrees(node_rs, he_rs, ones_col, z1)
    m1 = _tc_linear1(x, W1, b1.reshape(1, dh))
    h1s, dvis, dei = _tc_degree_scale(m1, dv_p, de_p)

    he_a = _sc_scatter_pass(h1s, node_rs, he_rs, z128)
    he1 = _tc_combine(he_a, dei)
    nb = _sc_scatter_pass(he1, he_rs, node_rs, z128)

    h2s = _tc_layer2(nb, dvis, w2p, b2p)

    he_a2 = _sc_scatter_pass(h2s, node_rs, he_rs, z48)
    he2 = _tc_combine(he_a2, dei)
    nb2 = _sc_scatter_pass(he2, he_rs, node_rs, z48)
    return _tc_combine(nb2, dvis, dout=nc)


# speedup vs baseline: 1.2698x; 1.2698x over previous
import functools

import jax
import jax.numpy as jnp
from jax import lax
from jax.experimental import pallas as pl
from jax.experimental.pallas import tpu as pltpu
from jax.experimental.pallas import tpu_sc as plsc

_NSC = 2
_NSUB = 16
_NW = _NSC * _NSUB
_CW = 125
_ROW_BLK = 2000

_mesh = plsc.VectorSubcoreMesh(core_axis_name="c", subcore_axis_name="s")
_SC_PARAMS = pltpu.CompilerParams(use_tc_tiling_on_sc=False)


_HW = 16


def _sc_degrees(nidx, hidx, ones_col, z1):
    nw, nch, cw = nidx.shape
    n = z1.shape[0]
    rps = (n // _NSUB) & ~7
    tail = n - rps * _NSUB

    @functools.partial(
        pl.kernel,
        out_type=(jax.ShapeDtypeStruct((_NSC, n, _HW), jnp.float32),
                  jax.ShapeDtypeStruct((_NSC, n, _HW), jnp.float32)),
        mesh=_mesh,
        scratch_types=[
            pltpu.VMEM((nch, cw), jnp.int32),
            pltpu.VMEM((nch, cw), jnp.int32),
            pltpu.VMEM((cw, _HW), jnp.float32),
            pltpu.VMEM_SHARED((n, _HW), jnp.float32),
            pltpu.VMEM_SHARED((n, _HW), jnp.float32),
        ],
        compiler_params=_SC_PARAMS,
    )
    def run(n_hbm, h_hbm, ones_hbm, z_hbm, dv_hbm, de_hbm,
            n_v, h_v, ones_v, dv_acc, de_acc):
        c = lax.axis_index("c")
        s = lax.axis_index("s")
        w = c * _NSUB + s
        pltpu.sync_copy(n_hbm.at[w], n_v)
        pltpu.sync_copy(h_hbm.at[w], h_v)
        pltpu.sync_copy(ones_hbm, ones_v)
        sl = pl.ds(s * rps, rps)
        pltpu.sync_copy(z_hbm.at[sl], dv_acc.at[sl])
        pltpu.sync_copy(z_hbm.at[sl], de_acc.at[sl])
        if tail:
            tl = pl.ds(rps * _NSUB, tail)

            @pl.when(s == 0)
            def _():
                pltpu.sync_copy(z_hbm.at[tl], dv_acc.at[tl])
                pltpu.sync_copy(z_hbm.at[tl], de_acc.at[tl])
        plsc.subcore_barrier()

        @pl.loop(0, nch)
        def _(j):
            pltpu.sync_copy(ones_v, dv_acc.at[n_v.at[j]], add=True)
            pltpu.sync_copy(ones_v, de_acc.at[h_v.at[j]], add=True)

        plsc.subcore_barrier()
        pltpu.sync_copy(dv_acc.at[sl], dv_hbm.at[c, sl])
        pltpu.sync_copy(de_acc.at[sl], de_hbm.at[c, sl])
        if tail:
            tl2 = pl.ds(rps * _NSUB, tail)

            @pl.when(s == 0)
            def _():
                pltpu.sync_copy(dv_acc.at[tl2], dv_hbm.at[c, tl2])
                pltpu.sync_copy(de_acc.at[tl2], de_hbm.at[c, tl2])

    return run(nidx, hidx, ones_col, z1)


def _sc_scatter_pass(tab, gidx, sidx, zeros):
    n, d = tab.shape
    nw, nch, cw = gidx.shape
    rps = (n // _NSUB) & ~7
    tail = n - rps * _NSUB
    nbuf = 2 if d > 64 else 4
    nhalves = 2 if d > 64 else 1
    nh = nch // nhalves
    assert nh * nhalves == nch and nh % nbuf == 0

    @functools.partial(
        pl.kernel,
        out_type=jax.ShapeDtypeStruct((_NSC, n, d), jnp.float32),
        mesh=_mesh,
        scratch_types=(
            [pltpu.VMEM((nh, cw), jnp.int32),
             pltpu.VMEM((nh, cw), jnp.int32)]
            + [pltpu.VMEM((cw, d), jnp.float32) for _ in range(nbuf)]
            + [pltpu.VMEM_SHARED((n, d), jnp.float32)]
            + [pltpu.SemaphoreType.DMA for _ in range(nbuf)]
        ),
        compiler_params=_SC_PARAMS,
    )
    def run(tab_hbm, g_hbm, s_hbm, z_hbm, out_hbm, g_v, s_v, *rest):
        bufs = rest[:nbuf]
        acc = rest[nbuf]
        sems = rest[nbuf + 1:]
        c = lax.axis_index("c")
        s = lax.axis_index("s")
        w = c * _NSUB + s
        sl = pl.ds(s * rps, rps)
        pltpu.sync_copy(z_hbm.at[sl], acc.at[sl])
        if tail:
            tl = pl.ds(rps * _NSUB, tail)

            @pl.when(s == 0)
            def _():
                pltpu.sync_copy(z_hbm.at[tl], acc.at[tl])
        plsc.subcore_barrier()

        for h in range(nhalves):
            pltpu.sync_copy(g_hbm.at[w, pl.ds(h * nh, nh)], g_v)
            pltpu.sync_copy(s_hbm.at[w, pl.ds(h * nh, nh)], s_v)
            for b in range(nbuf):
                pltpu.async_copy(tab_hbm.at[g_v.at[b]], bufs[b], sems[b])

            @pl.loop(0, nh, step=nbuf)
            def _(j):
                for b in range(nbuf):
                    pltpu.make_async_copy(tab_hbm.at[pl.ds(0, cw)], bufs[b],
                                          sems[b]).wait()
                    pltpu.sync_copy(bufs[b], acc.at[s_v.at[j + b]], add=True)

                    @pl.when(j + nbuf + b < nh)
                    def _(b=b):
                        pltpu.async_copy(tab_hbm.at[g_v.at[j + nbuf + b]],
                                         bufs[b], sems[b])

        plsc.subcore_barrier()
        pltpu.sync_copy(acc.at[sl], out_hbm.at[c, sl])
        if tail:
            tl2 = pl.ds(rps * _NSUB, tail)

            @pl.when(s == 0)
            def _():
                pltpu.sync_copy(acc.at[tl2], out_hbm.at[c, tl2])

    return run(tab, gidx, sidx, zeros)


def _lin1_body(x_ref, w_ref, b_ref, o_ref):
    o_ref[...] = (jnp.dot(x_ref[...], w_ref[...],
                          preferred_element_type=jnp.float32) + b_ref[...])


def _tc_linear1(x, w, b):
    n, din = x.shape
    dh = w.shape[1]
    return pl.pallas_call(
        _lin1_body,
        grid=(n // _ROW_BLK,),
        in_specs=[pl.BlockSpec((_ROW_BLK, din), lambda i: (i, 0)),
                  pl.BlockSpec((din, dh), lambda i: (0, 0)),
                  pl.BlockSpec((1, dh), lambda i: (0, 0))],
        out_specs=pl.BlockSpec((_ROW_BLK, dh), lambda i: (i, 0)),
        out_shape=jax.ShapeDtypeStruct((n, dh), jnp.float32),
    )(x, w, b)


def _deg_body(m_ref, dv_ref, de_ref, h_ref, dvis_ref, dei_ref):
    dv = dv_ref[0, :, :1] + dv_ref[1, :, :1]
    dvis = jnp.where(dv > 0, lax.rsqrt(jnp.maximum(dv, 1e-12)), 0.0)
    de = de_ref[0, :, :1] + de_ref[1, :, :1]
    dei = jnp.where(de > 0, 1.0 / jnp.maximum(de, 1e-12), 0.0)
    dvis_ref[...] = dvis
    dei_ref[...] = dei
    h_ref[...] = m_ref[...] * dvis


def _tc_degree_scale(m1, dv_p, de_p):
    n, dh = m1.shape
    col = lambda i: (i, 0)
    pspec = pl.BlockSpec((_NSC, _ROW_BLK, _HW), lambda i: (0, i, 0))
    cspec = pl.BlockSpec((_ROW_BLK, 1), col)
    return pl.pallas_call(
        _deg_body,
        grid=(n // _ROW_BLK,),
        in_specs=[pl.BlockSpec((_ROW_BLK, dh), col), pspec, pspec],
        out_specs=[pl.BlockSpec((_ROW_BLK, dh), col), cspec, cspec],
        out_shape=[jax.ShapeDtypeStruct((n, dh), jnp.float32),
                   jax.ShapeDtypeStruct((n, 1), jnp.float32),
                   jax.ShapeDtypeStruct((n, 1), jnp.float32)],
    )(m1, dv_p, de_p)


def _comb_body(p_ref, sc_ref, o_ref):
    o_ref[...] = (p_ref[0] + p_ref[1]) * sc_ref[...]


def _tc_combine(p, scale, dout=None):
    _, n, d = p.shape
    do = d if dout is None else dout
    col = lambda i: (i, 0)

    def body(p_ref, sc_ref, o_ref):
        o_ref[...] = ((p_ref[0] + p_ref[1]) * sc_ref[...])[:, :do]

    return pl.pallas_call(
        body,
        grid=(n // _ROW_BLK,),
        in_specs=[pl.BlockSpec((_NSC, _ROW_BLK, d), lambda i: (0, i, 0)),
                  pl.BlockSpec((_ROW_BLK, 1), col)],
        out_specs=pl.BlockSpec((_ROW_BLK, do), col),
        out_shape=jax.ShapeDtypeStruct((n, do), jnp.float32),
    )(p, scale)


def _layer2_body(p_ref, dvis_ref, w_ref, b_ref, o_ref):
    t = jax.nn.relu((p_ref[0] + p_ref[1]) * dvis_ref[...])
    o_ref[...] = (jnp.dot(t, w_ref[...], preferred_element_type=jnp.float32)
                  + b_ref[...]) * dvis_ref[...]


def _tc_layer2(p, dvis, w2p, b2p):
    _, n, dh = p.shape
    d2 = w2p.shape[1]
    col = lambda i: (i, 0)
    return pl.pallas_call(
        _layer2_body,
        grid=(n // _ROW_BLK,),
        in_specs=[pl.BlockSpec((_NSC, _ROW_BLK, dh), lambda i: (0, i, 0)),
                  pl.BlockSpec((_ROW_BLK, 1), col),
                  pl.BlockSpec((dh, d2), lambda i: (0, 0)),
                  pl.BlockSpec((1, d2), lambda i: (0, 0))],
        out_specs=pl.BlockSpec((_ROW_BLK, d2), col),
        out_shape=jax.ShapeDtypeStruct((n, d2), jnp.float32),
    )(p, dvis, w2p, b2p)


def kernel(x, hyperedge_index, W1, b1, W2, b2):
    n, din = x.shape
    dh = W1.shape[1]
    nc = W2.shape[1]
    e = hyperedge_index.shape[1]
    per = e // _NW
    nch = per // _CW
    assert per * _NW == e and nch * _CW == per and nch % 2 == 0

    node_rs = hyperedge_index[0].reshape(_NW, nch, _CW)
    he_rs = hyperedge_index[1].reshape(_NW, nch, _CW)

    d2 = 48
    z128 = jnp.zeros((n, dh), jnp.float32)
    z48 = jnp.zeros((n, d2), jnp.float32)
    z1 = jnp.zeros((n, _HW), jnp.float32)
    ones_col = jnp.ones((_CW, _HW), jnp.float32)
    w2p = jnp.pad(W2, ((0, 0), (0, d2 - nc)))
    b2p = jnp.pad(b2, (0, d2 - nc)).reshape(1, d2)

    dv_p, de_p = _sc_degrees(node_rs, he_rs, ones_col, z1)
    m1 = _tc_linear1(x, W1, b1.reshape(1, dh))
    h1s, dvis, dei = _tc_degree_scale(m1, dv_p, de_p)

    he_a = _sc_scatter_pass(h1s, node_rs, he_rs, z128)
    he1 = _tc_combine(he_a, dei)
    nb = _sc_scatter_pass(he1, he_rs, node_rs, z128)

    h2s = _tc_layer2(nb, dvis, w2p, b2p)

    he_a2 = _sc_scatter_pass(h2s, node_rs, he_rs, z48)
    he2 = _tc_combine(he_a2, dei)
    nb2 = _sc_scatter_pass(he2, he_rs, node_rs, z48)
    return _tc_combine(nb2, dvis, dout=nc)


# speedup vs baseline: 1.2720x; 1.0017x over previous
import functools

import jax
import jax.numpy as jnp
from jax import lax
from jax.experimental import pallas as pl
from jax.experimental.pallas import tpu as pltpu
from jax.experimental.pallas import tpu_sc as plsc

_NSC = 2
_NSUB = 16
_NW = _NSC * _NSUB
_CW = 125
_ROW_BLK = 2000

_mesh = plsc.VectorSubcoreMesh(core_axis_name="c", subcore_axis_name="s")
_SC_PARAMS = pltpu.CompilerParams(use_tc_tiling_on_sc=False)


_HW = 16


def _sc_degrees(nidx, hidx, ones_col, z1):
    nw, nch, cw = nidx.shape
    n = z1.shape[0]
    rps = (n // _NSUB) & ~7
    tail = n - rps * _NSUB

    @functools.partial(
        pl.kernel,
        out_type=(jax.ShapeDtypeStruct((_NSC, n, _HW), jnp.float32),
                  jax.ShapeDtypeStruct((_NSC, n, _HW), jnp.float32)),
        mesh=_mesh,
        scratch_types=[
            pltpu.VMEM((nch, cw), jnp.int32),
            pltpu.VMEM((nch, cw), jnp.int32),
            pltpu.VMEM((cw, _HW), jnp.float32),
            pltpu.VMEM_SHARED((n, _HW), jnp.float32),
            pltpu.VMEM_SHARED((n, _HW), jnp.float32),
        ],
        compiler_params=_SC_PARAMS,
    )
    def run(n_hbm, h_hbm, ones_hbm, z_hbm, dv_hbm, de_hbm,
            n_v, h_v, ones_v, dv_acc, de_acc):
        c = lax.axis_index("c")
        s = lax.axis_index("s")
        w = c * _NSUB + s
        pltpu.sync_copy(n_hbm.at[w], n_v)
        pltpu.sync_copy(h_hbm.at[w], h_v)
        pltpu.sync_copy(ones_hbm, ones_v)
        sl = pl.ds(s * rps, rps)
        pltpu.sync_copy(z_hbm.at[sl], dv_acc.at[sl])
        pltpu.sync_copy(z_hbm.at[sl], de_acc.at[sl])
        if tail:
            tl = pl.ds(rps * _NSUB, tail)

            @pl.when(s == 0)
            def _():
                pltpu.sync_copy(z_hbm.at[tl], dv_acc.at[tl])
                pltpu.sync_copy(z_hbm.at[tl], de_acc.at[tl])
        plsc.subcore_barrier()

        @pl.loop(0, nch)
        def _(j):
            pltpu.sync_copy(ones_v, dv_acc.at[n_v.at[j]], add=True)
            pltpu.sync_copy(ones_v, de_acc.at[h_v.at[j]], add=True)

        plsc.subcore_barrier()
        pltpu.sync_copy(dv_acc.at[sl], dv_hbm.at[c, sl])
        pltpu.sync_copy(de_acc.at[sl], de_hbm.at[c, sl])
        if tail:
            tl2 = pl.ds(rps * _NSUB, tail)

            @pl.when(s == 0)
            def _():
                pltpu.sync_copy(dv_acc.at[tl2], dv_hbm.at[c, tl2])
                pltpu.sync_copy(de_acc.at[tl2], de_hbm.at[c, tl2])

    return run(nidx, hidx, ones_col, z1)


def _sc_scatter_pass(tab, gidx, sidx, zeros):
    n, d = tab.shape
    nw, nch, cw = gidx.shape
    rps = (n // _NSUB) & ~7
    tail = n - rps * _NSUB
    nbuf = 2 if d > 64 else 4
    nhalves = 2 if d > 64 else 1
    nh = nch // nhalves
    assert nh * nhalves == nch and nh % nbuf == 0

    @functools.partial(
        pl.kernel,
        out_type=jax.ShapeDtypeStruct((_NSC, n, d), jnp.float32),
        mesh=_mesh,
        scratch_types=(
            [pltpu.VMEM((nh, cw), jnp.int32),
             pltpu.VMEM((nh, cw), jnp.int32)]
            + [pltpu.VMEM((cw, d), jnp.float32) for _ in range(nbuf)]
            + [pltpu.VMEM_SHARED((n, d), jnp.float32)]
            + [pltpu.SemaphoreType.DMA for _ in range(nbuf)]
        ),
        compiler_params=_SC_PARAMS,
    )
    def run(tab_hbm, g_hbm, s_hbm, z_hbm, out_hbm, g_v, s_v, *rest):
        bufs = rest[:nbuf]
        acc = rest[nbuf]
        sems = rest[nbuf + 1:]
        c = lax.axis_index("c")
        s = lax.axis_index("s")
        w = c * _NSUB + s
        sl = pl.ds(s * rps, rps)
        pltpu.sync_copy(z_hbm.at[sl], acc.at[sl])
        if tail:
            tl = pl.ds(rps * _NSUB, tail)

            @pl.when(s == 0)
            def _():
                pltpu.sync_copy(z_hbm.at[tl], acc.at[tl])
        plsc.subcore_barrier()

        for h in range(nhalves):
            pltpu.sync_copy(g_hbm.at[w, pl.ds(h * nh, nh)], g_v)
            pltpu.sync_copy(s_hbm.at[w, pl.ds(h * nh, nh)], s_v)
            for b in range(nbuf):
                pltpu.async_copy(tab_hbm.at[g_v.at[b]], bufs[b], sems[b])

            @pl.loop(0, nh, step=nbuf)
            def _(j):
                for b in range(nbuf):
                    pltpu.make_async_copy(tab_hbm.at[pl.ds(0, cw)], bufs[b],
                                          sems[b]).wait()
                    pltpu.sync_copy(bufs[b], acc.at[s_v.at[j + b]], add=True)

                    @pl.when(j + nbuf + b < nh)
                    def _(b=b):
                        pltpu.async_copy(tab_hbm.at[g_v.at[j + nbuf + b]],
                                         bufs[b], sems[b])

        plsc.subcore_barrier()
        pltpu.sync_copy(acc.at[sl], out_hbm.at[c, sl])
        if tail:
            tl2 = pl.ds(rps * _NSUB, tail)

            @pl.when(s == 0)
            def _():
                pltpu.sync_copy(acc.at[tl2], out_hbm.at[c, tl2])

    return run(tab, gidx, sidx, zeros)


def _lin1_body(x_ref, w_ref, b_ref, dv_ref, de_ref,
               h_ref, dvis_ref, dei_ref):
    dv = dv_ref[0, :, :1] + dv_ref[1, :, :1]
    dvis = jnp.where(dv > 0, lax.rsqrt(jnp.maximum(dv, 1e-12)), 0.0)
    de = de_ref[0, :, :1] + de_ref[1, :, :1]
    dei = jnp.where(de > 0, 1.0 / jnp.maximum(de, 1e-12), 0.0)
    dvis_ref[...] = dvis
    dei_ref[...] = dei
    m = (jnp.dot(x_ref[...], w_ref[...],
                 preferred_element_type=jnp.float32) + b_ref[...])
    h_ref[...] = m * dvis


def _tc_linear1_scale(x, w, b, dv_p, de_p):
    n, din = x.shape
    dh = w.shape[1]
    col = lambda i: (i, 0)
    pspec = pl.BlockSpec((_NSC, _ROW_BLK, _HW), lambda i: (0, i, 0))
    cspec = pl.BlockSpec((_ROW_BLK, 1), col)
    return pl.pallas_call(
        _lin1_body,
        grid=(n // _ROW_BLK,),
        in_specs=[pl.BlockSpec((_ROW_BLK, din), col),
                  pl.BlockSpec((din, dh), lambda i: (0, 0)),
                  pl.BlockSpec((1, dh), lambda i: (0, 0)),
                  pspec, pspec],
        out_specs=[pl.BlockSpec((_ROW_BLK, dh), col), cspec, cspec],
        out_shape=[jax.ShapeDtypeStruct((n, dh), jnp.float32),
                   jax.ShapeDtypeStruct((n, 1), jnp.float32),
                   jax.ShapeDtypeStruct((n, 1), jnp.float32)],
    )(x, w, b, dv_p, de_p)


def _comb_body(p_ref, sc_ref, o_ref):
    o_ref[...] = (p_ref[0] + p_ref[1]) * sc_ref[...]


def _tc_combine(p, scale, dout=None):
    _, n, d = p.shape
    do = d if dout is None else dout
    col = lambda i: (i, 0)

    def body(p_ref, sc_ref, o_ref):
        o_ref[...] = ((p_ref[0] + p_ref[1]) * sc_ref[...])[:, :do]

    return pl.pallas_call(
        body,
        grid=(n // _ROW_BLK,),
        in_specs=[pl.BlockSpec((_NSC, _ROW_BLK, d), lambda i: (0, i, 0)),
                  pl.BlockSpec((_ROW_BLK, 1), col)],
        out_specs=pl.BlockSpec((_ROW_BLK, do), col),
        out_shape=jax.ShapeDtypeStruct((n, do), jnp.float32),
    )(p, scale)


def _layer2_body(p_ref, dvis_ref, w_ref, b_ref, o_ref):
    t = jax.nn.relu((p_ref[0] + p_ref[1]) * dvis_ref[...])
    o_ref[...] = (jnp.dot(t, w_ref[...], preferred_element_type=jnp.float32)
                  + b_ref[...]) * dvis_ref[...]


def _tc_layer2(p, dvis, w2p, b2p):
    _, n, dh = p.shape
    d2 = w2p.shape[1]
    col = lambda i: (i, 0)
    return pl.pallas_call(
        _layer2_body,
        grid=(n // _ROW_BLK,),
        in_specs=[pl.BlockSpec((_NSC, _ROW_BLK, dh), lambda i: (0, i, 0)),
                  pl.BlockSpec((_ROW_BLK, 1), col),
                  pl.BlockSpec((dh, d2), lambda i: (0, 0)),
                  pl.BlockSpec((1, d2), lambda i: (0, 0))],
        out_specs=pl.BlockSpec((_ROW_BLK, d2), col),
        out_shape=jax.ShapeDtypeStruct((n, d2), jnp.float32),
    )(p, dvis, w2p, b2p)


def kernel(x, hyperedge_index, W1, b1, W2, b2):
    n, din = x.shape
    dh = W1.shape[1]
    nc = W2.shape[1]
    e = hyperedge_index.shape[1]
    per = e // _NW
    nch = per // _CW
    assert per * _NW == e and nch * _CW == per and nch % 2 == 0

    node_rs = hyperedge_index[0].reshape(_NW, nch, _CW)
    he_rs = hyperedge_index[1].reshape(_NW, nch, _CW)

    d2 = 48
    z128 = jnp.zeros((n, dh), jnp.float32)
    z48 = jnp.zeros((n, d2), jnp.float32)
    z1 = jnp.zeros((n, _HW), jnp.float32)
    ones_col = jnp.ones((_CW, _HW), jnp.float32)
    w2p = jnp.pad(W2, ((0, 0), (0, d2 - nc)))
    b2p = jnp.pad(b2, (0, d2 - nc)).reshape(1, d2)

    dv_p, de_p = _sc_degrees(node_rs, he_rs, ones_col, z1)
    h1s, dvis, dei = _tc_linear1_scale(x, W1, b1.reshape(1, dh), dv_p, de_p)

    he_a = _sc_scatter_pass(h1s, node_rs, he_rs, z128)
    he1 = _tc_combine(he_a, dei)
    nb = _sc_scatter_pass(he1, he_rs, node_rs, z128)

    h2s = _tc_layer2(nb, dvis, w2p, b2p)

    he_a2 = _sc_scatter_pass(h2s, node_rs, he_rs, z48)
    he2 = _tc_combine(he_a2, dei)
    nb2 = _sc_scatter_pass(he2, he_rs, node_rs, z48)
    return _tc_combine(nb2, dvis, dout=nc)
